# baseline (speedup 1.0000x reference)
import jax
import jax.numpy as jnp
from jax import lax
from jax.experimental import pallas as pl
from jax.experimental.pallas import tpu as pltpu
from jax.experimental.pallas import tpu_sc as plsc

_N_ROWS = 16384
_N_ATOMS = 200
_NC = 2
_NS = 16
_NW = _NC * _NS
_ROWS_PER_W = _N_ROWS // _NW
_CHUNK_ROWS = 128
_N_CHUNKS = _ROWS_PER_W // _CHUNK_ROWS
_GROUPS_PER_CHUNK = _CHUNK_ROWS // 16
_QUARTER = _N_ATOMS // 4


def _sc_body(species_hbm, energies_hbm, se_hbm, out_hbm,
             spec0, spec1, en_v, out_v, table_v, sem0, sem1):
  cid = lax.axis_index("c")
  sid = lax.axis_index("s")
  wid = sid * _NC + cid
  row0 = wid * _ROWS_PER_W

  pltpu.sync_copy(se_hbm, table_v)
  pltpu.sync_copy(energies_hbm.at[pl.ds(row0, _ROWS_PER_W)], en_v)

  bufs = (spec0, spec1)
  sems = (sem0, sem1)
  lanes = lax.iota(jnp.int32, 16)

  def start_in(k):
    src = species_hbm.at[
        pl.ds((row0 + k * _CHUNK_ROWS) * _N_ATOMS, _CHUNK_ROWS * _N_ATOMS)]
    return pltpu.async_copy(src, bufs[k % 2], sems[k % 2])

  handles = {0: start_in(0)}
  for k in range(_N_CHUNKS):
    handles[k].wait()
    if k + 1 < _N_CHUNKS:
      handles[k + 1] = start_in(k + 1)
    spec_v = bufs[k % 2]

    def group_body(g, carry, spec_v=spec_v, k=k):
      rbase = (g * 16 + lanes) * _N_ATOMS

      def jbody(j, st):
        a0, a1, a2, a3, i0, i1, i2, i3 = st
        a0 = a0 + plsc.load_gather(table_v, [plsc.load_gather(spec_v, [i0])])
        a1 = a1 + plsc.load_gather(table_v, [plsc.load_gather(spec_v, [i1])])
        a2 = a2 + plsc.load_gather(table_v, [plsc.load_gather(spec_v, [i2])])
        a3 = a3 + plsc.load_gather(table_v, [plsc.load_gather(spec_v, [i3])])
        return (a0, a1, a2, a3, i0 + 1, i1 + 1, i2 + 1, i3 + 1)

      z = jnp.zeros((16,), jnp.float32)
      st = lax.fori_loop(
          0, _QUARTER, jbody,
          (z, z, z, z,
           rbase, rbase + _QUARTER, rbase + 2 * _QUARTER, rbase + 3 * _QUARTER),
          unroll=5)
      acc = (st[0] + st[1]) + (st[2] + st[3])
      off = pl.multiple_of(k * _CHUNK_ROWS + g * 16, 16)
      out_v[pl.ds(off, 16)] = acc + en_v[pl.ds(off, 16)]
      return carry

    lax.fori_loop(0, _GROUPS_PER_CHUNK, group_body, 0)

  pltpu.sync_copy(out_v, out_hbm.at[pl.ds(row0, _ROWS_PER_W)])


def _tc_copy_body(x_ref, o_ref):
  o_ref[...] = x_ref[...]


@jax.jit
def _shifted(species, energies, self_energies):
  mesh = plsc.VectorSubcoreMesh(core_axis_name="c", subcore_axis_name="s")
  sc_fn = pl.kernel(
      _sc_body,
      out_type=jax.ShapeDtypeStruct((_N_ROWS,), jnp.float32),
      mesh=mesh,
      compiler_params=pltpu.CompilerParams(needs_layout_passes=False),
      scratch_types=[
          pltpu.VMEM((_CHUNK_ROWS * _N_ATOMS,), jnp.int32),
          pltpu.VMEM((_CHUNK_ROWS * _N_ATOMS,), jnp.int32),
          pltpu.VMEM((_ROWS_PER_W,), jnp.float32),
          pltpu.VMEM((_ROWS_PER_W,), jnp.float32),
          pltpu.VMEM((10,), jnp.float32),
          pltpu.SemaphoreType.DMA,
          pltpu.SemaphoreType.DMA,
      ],
  )
  flat = species.reshape(-1)
  out_en = sc_fn(flat, energies, self_energies)

  spec2d = flat.reshape(1600, 2048)
  copied = pl.pallas_call(
      _tc_copy_body,
      grid=(8,),
      in_specs=[pl.BlockSpec((200, 2048), lambda i: (i, 0))],
      out_specs=pl.BlockSpec((200, 2048), lambda i: (i, 0)),
      out_shape=jax.ShapeDtypeStruct((1600, 2048), jnp.int32),
  )(spec2d)
  return copied.reshape(_N_ROWS, _N_ATOMS), out_en


def kernel(species, energies, self_energies):
  copied, out = _shifted(species.astype(jnp.int32), energies, self_energies)
  return (copied.astype(species.dtype), out.astype(energies.dtype))
